# Initial kernel scaffold; baseline (speedup 1.0000x reference)
import jax
import jax.numpy as jnp
from jax.experimental import pallas as pl


def kernel(x, keys, vals, W_q, gate_w, gate_b):
    raise NotImplementedError("write your pallas kernel here")


# speedup vs baseline: 25.1006x; 25.1006x over previous
import functools

import jax
import jax.numpy as jnp
from jax import lax
from jax.experimental import pallas as pl
from jax.experimental.pallas import tpu as pltpu
from jax.experimental.pallas import tpu_sc as plsc

D_MODEL = 512
NUM_SLOTS = 65536
SEQ = 2048
K_TOP = 32
GRP = 32
NGRP = NUM_SLOTS // GRP
ALPHA = 1.0
TAU = 10.0
NEG = -3.0e38

SC_NC = 2
SC_NS = 16
SC_NW = SC_NC * SC_NS

def _qn_body(x_ref, w_ref, qn_ref):
    q = lax.dot_general(x_ref[...], w_ref[...], (((1,), (1,)), ((), ())),
                        preferred_element_type=jnp.float32)
    n = jnp.sqrt(jnp.sum(q * q, axis=1, keepdims=True))
    qn_ref[...] = q / jnp.maximum(n, 1e-12)


QB = 256
KB = 4096

def _scores_body(keys_ref, qn_ref, s_ref, pooled_ref, knb_ref):
    @pl.when(pl.program_id(1) == 0)
    def _():
        kb = keys_ref[...]
        n = jnp.sqrt(jnp.sum(kb * kb, axis=1, keepdims=True))
        knb_ref[...] = kb / jnp.maximum(n, 1e-12)

    s = lax.dot_general(qn_ref[...], knb_ref[...], (((1,), (1,)), ((), ())),
                        preferred_element_type=jnp.float32)
    s = s / jnp.float32(D_MODEL ** 0.5)
    s_ref[...] = s
    pooled_ref[...] = jnp.max(s.reshape(QB, KB // GRP, GRP), axis=-1)


QB2 = 256

def _grpsel_body(pooled_ref, blkidx_ref, eids_ref, p_ref):
    p_ref[...] = pooled_ref[...]
    io = lax.broadcasted_iota(jnp.int32, (QB2, NGRP), 1)
    iob = lax.broadcasted_iota(jnp.int32, (QB2, BLKW), 1)
    iok = lax.broadcasted_iota(jnp.int32, (QB2, K_TOP), 1)
    rows = pl.program_id(0) * QB2 + lax.broadcasted_iota(jnp.int32, (QB2, 1), 0)

    def pass_body(j, bi):
        p = p_ref[...]
        cur = jnp.max(p, axis=1, keepdims=True)
        g = jnp.min(jnp.where(p == cur, io, NGRP), axis=1, keepdims=True)
        blk = lax.shift_right_logical(g, 2)
        bi = jnp.where(iok == j, rows * NBLK + blk, bi)
        eids_ref[:, pl.ds(j * BLKW, BLKW)] = blk * BLKW + iob
        p_ref[...] = jnp.where(io == g, NEG, p)
        return bi

    blkidx_ref[...] = lax.fori_loop(
        0, K_TOP, pass_body, jnp.zeros((QB2, K_TOP), jnp.int32))


BLKW = 128
NBLK = NUM_SLOTS // BLKW
C_ROWS = SEQ * K_TOP
C_PER_W = C_ROWS // SC_NW
C_CHUNK = 128

def _cand_gather_body(tab_hbm, blk_hbm, out_hbm, blkidx_v, rows_v, sem):
    wid = lax.axis_index("s") * SC_NC + lax.axis_index("c")
    base = wid * C_PER_W

    def chunk_body(ch, carry):
        off = base + ch * C_CHUNK
        pltpu.sync_copy(blk_hbm.at[pl.ds(off, C_CHUNK)], blkidx_v)
        pltpu.async_copy(tab_hbm.at[blkidx_v], rows_v, sem).wait()
        pltpu.sync_copy(rows_v, out_hbm.at[pl.ds(off, C_CHUNK)])
        return carry

    lax.fori_loop(0, C_PER_W // C_CHUNK, chunk_body, 0)


NCAND = K_TOP * BLKW

def _topk_body(c_ref, e_ref, x_ref, gw_ref, gb_ref,
               topi_ref, wts_ref, c_s):
    c_s[...] = c_ref[...]
    big = jnp.int32(NUM_SLOTS)
    iok = lax.broadcasted_iota(jnp.int32, (QB2, K_TOP), 1)

    def pass_body(j, carry):
        ti, tvv = carry
        c = c_s[...]
        e = e_ref[...]
        cur = jnp.max(c, axis=1, keepdims=True)
        gid = jnp.min(jnp.where(c == cur, e, big), axis=1, keepdims=True)
        ti = jnp.where(iok == j, gid, ti)
        tvv = jnp.where(iok == j, cur, tvv)
        c_s[...] = jnp.where(e == gid, NEG, c)
        return ti, tvv

    ti, tv = lax.fori_loop(
        0, K_TOP, pass_body,
        (jnp.zeros((QB2, K_TOP), jnp.int32),
         jnp.zeros((QB2, K_TOP), jnp.float32)))
    topi_ref[...] = ti
    a = jnp.exp((tv - tv[:, 0:1]) / jnp.float32(TAU))
    attn = a / jnp.sum(a, axis=1, keepdims=True)
    glin = lax.dot_general(x_ref[...], gw_ref[...], (((1,), (1,)), ((), ())),
                           preferred_element_type=jnp.float32)[:, 0:1]
    gate = jax.nn.sigmoid(glin + gb_ref[0, 0])
    w = attn * gate * jnp.float32(ALPHA)
    for j in range(K_TOP):
        wts_ref[:, j * 16:(j + 1) * 16] = jnp.broadcast_to(
            w[:, j:j + 1], (QB2, 16))


E_PER_W = SEQ // SC_NW
DCH = D_MODEL // 16

def _combine_body(vals_hbm, topi_hbm, wts_hbm, out_hbm,
                  idx_v, w_v, rows_v, orow_v, sem):
    wid = lax.axis_index("s") * SC_NC + lax.axis_index("c")
    base = wid * E_PER_W

    def row_body(i, carry):
        r = base + i
        pltpu.sync_copy(topi_hbm.at[r], idx_v)
        pltpu.sync_copy(wts_hbm.at[r], w_v)
        pltpu.async_copy(vals_hbm.at[idx_v], rows_v, sem).wait()
        wb = [w_v[pl.ds(k * 16, 16)] for k in range(K_TOP)]
        for dc in range(DCH):
            acc = wb[0] * rows_v[0, pl.ds(dc * 16, 16)]
            for k in range(1, K_TOP):
                acc = acc + wb[k] * rows_v[k, pl.ds(dc * 16, 16)]
            orow_v[pl.ds(dc * 16, 16)] = acc
        pltpu.sync_copy(orow_v, out_hbm.at[r])
        return carry

    lax.fori_loop(0, E_PER_W, row_body, 0)


@jax.jit
def kernel(x, keys, vals, W_q, gate_w, gate_b):
    B, L, D = x.shape
    x2d = x.reshape(L, D)

    qn = pl.pallas_call(
        _qn_body,
        out_shape=jax.ShapeDtypeStruct((L, D), jnp.float32),
        in_specs=[pl.BlockSpec((L, D), lambda: (0, 0)),
                  pl.BlockSpec((D, D), lambda: (0, 0))],
        out_specs=pl.BlockSpec((L, D), lambda: (0, 0)),
    )(x2d, W_q)

    nk = NUM_SLOTS // KB
    nq = L // QB
    scores, pooled = pl.pallas_call(
        _scores_body,
        grid=(nk, nq),
        out_shape=[jax.ShapeDtypeStruct((L, NUM_SLOTS), jnp.float32),
                   jax.ShapeDtypeStruct((L, NGRP), jnp.float32)],
        in_specs=[pl.BlockSpec((KB, D), lambda k, q: (k, 0)),
                  pl.BlockSpec((QB, D), lambda k, q: (q, 0))],
        out_specs=[pl.BlockSpec((QB, KB), lambda k, q: (q, k)),
                   pl.BlockSpec((QB, KB // GRP), lambda k, q: (q, k))],
        scratch_shapes=[pltpu.VMEM((KB, D), jnp.float32)],
    )(keys, qn)

    blkidx, eids = pl.pallas_call(
        _grpsel_body,
        grid=(L // QB2,),
        out_shape=[jax.ShapeDtypeStruct((L, K_TOP), jnp.int32),
                   jax.ShapeDtypeStruct((L, NCAND), jnp.int32)],
        in_specs=[pl.BlockSpec((QB2, NGRP), lambda i: (i, 0))],
        out_specs=[pl.BlockSpec((QB2, K_TOP), lambda i: (i, 0)),
                   pl.BlockSpec((QB2, NCAND), lambda i: (i, 0))],
        scratch_shapes=[pltpu.VMEM((QB2, NGRP), jnp.float32)],
    )(pooled)

    mesh = plsc.VectorSubcoreMesh(core_axis_name="c", subcore_axis_name="s")
    cand = pl.kernel(
        _cand_gather_body,
        out_type=jax.ShapeDtypeStruct((C_ROWS, BLKW), jnp.float32),
        mesh=mesh,
        scratch_types=[pltpu.VMEM((C_CHUNK,), jnp.int32),
                       pltpu.VMEM((C_CHUNK, BLKW), jnp.float32),
                       pltpu.SemaphoreType.DMA],
    )(scores.reshape(L * NBLK, BLKW), blkidx.reshape(C_ROWS))

    gw8 = jnp.broadcast_to(gate_w.reshape(1, D), (8, D))
    gb8 = jnp.broadcast_to(gate_b.reshape(1, 1), (8, 128))
    topi, wts = pl.pallas_call(
        _topk_body,
        grid=(L // QB2,),
        out_shape=[jax.ShapeDtypeStruct((L, K_TOP), jnp.int32),
                   jax.ShapeDtypeStruct((L, K_TOP * 16), jnp.float32)],
        in_specs=[pl.BlockSpec((QB2, NCAND), lambda i: (i, 0)),
                  pl.BlockSpec((QB2, NCAND), lambda i: (i, 0)),
                  pl.BlockSpec((QB2, D), lambda i: (i, 0)),
                  pl.BlockSpec((8, D), lambda i: (0, 0)),
                  pl.BlockSpec((8, 128), lambda i: (0, 0))],
        out_specs=[pl.BlockSpec((QB2, K_TOP), lambda i: (i, 0)),
                   pl.BlockSpec((QB2, K_TOP * 16), lambda i: (i, 0))],
        scratch_shapes=[pltpu.VMEM((QB2, NCAND), jnp.float32)],
    )(cand.reshape(L, NCAND), eids, x2d, gw8, gb8)

    out2d = pl.kernel(
        _combine_body,
        out_type=jax.ShapeDtypeStruct((L, D), jnp.float32),
        mesh=mesh,
        scratch_types=[pltpu.VMEM((K_TOP,), jnp.int32),
                       pltpu.VMEM((K_TOP * 16,), jnp.float32),
                       pltpu.VMEM((K_TOP, D), jnp.float32),
                       pltpu.VMEM((D,), jnp.float32),
                       pltpu.SemaphoreType.DMA],
    )(vals, topi, wts)

    return out2d.reshape(B, L, D)


# speedup vs baseline: 36.7007x; 1.4621x over previous
import functools

import jax
import jax.numpy as jnp
from jax import lax
from jax.experimental import pallas as pl
from jax.experimental.pallas import tpu as pltpu
from jax.experimental.pallas import tpu_sc as plsc

D_MODEL = 512
NUM_SLOTS = 65536
SEQ = 2048
K_TOP = 32
GRP = 128
NGRP = NUM_SLOTS // GRP
ALPHA = 1.0
TAU = 10.0
NEG = -3.0e38

SC_NC = 2
SC_NS = 16
SC_NW = SC_NC * SC_NS

def _qn_body(x_ref, w_ref, qn_ref):
    q = lax.dot_general(x_ref[...], w_ref[...], (((1,), (1,)), ((), ())),
                        preferred_element_type=jnp.float32)
    n = jnp.sqrt(jnp.sum(q * q, axis=1, keepdims=True))
    qn_ref[...] = q / jnp.maximum(n, 1e-12)


QB = 256
KB = 4096

def _scores_body(keys_ref, qn_ref, s_ref, pooled_ref, knb_ref):
    @pl.when(pl.program_id(1) == 0)
    def _():
        kb = keys_ref[...]
        n = jnp.sqrt(jnp.sum(kb * kb, axis=1, keepdims=True))
        knb_ref[...] = kb / jnp.maximum(n, 1e-12)

    s = lax.dot_general(qn_ref[...], knb_ref[...], (((1,), (1,)), ((), ())),
                        preferred_element_type=jnp.float32)
    s = s / jnp.float32(D_MODEL ** 0.5)
    s_ref[...] = s
    pooled_ref[0] = jnp.max(s.reshape(QB, KB // GRP, GRP), axis=-1)


QB2 = 256

def _grpsel_body(pooled_ref, blkidx_ref, eids_ref, p_ref):
    p_ref[...] = pooled_ref[...].transpose(1, 0, 2).reshape(QB2, NGRP)
    io = lax.broadcasted_iota(jnp.int32, (QB2, NGRP), 1)
    iob = lax.broadcasted_iota(jnp.int32, (QB2, BLKW), 1)
    iok = lax.broadcasted_iota(jnp.int32, (QB2, K_TOP), 1)
    rows = pl.program_id(0) * QB2 + lax.broadcasted_iota(jnp.int32, (QB2, 1), 0)

    def pass_body(j, bi):
        p = p_ref[...]
        cur = jnp.max(p, axis=1, keepdims=True)
        g = jnp.min(jnp.where(p == cur, io, NGRP), axis=1, keepdims=True)
        bi = jnp.where(iok == j, rows * NBLK + g, bi)
        blk = g
        eids_ref[:, pl.ds(j * BLKW, BLKW)] = blk * BLKW + iob
        p_ref[...] = jnp.where(io == g, NEG, p)
        return bi

    blkidx_ref[...] = lax.fori_loop(
        0, K_TOP, pass_body, jnp.zeros((QB2, K_TOP), jnp.int32))


BLKW = 128
NBLK = NUM_SLOTS // BLKW
C_ROWS = SEQ * K_TOP
C_PER_W = C_ROWS // SC_NW
C_CHUNK = 128

def _cand_gather_body(tab_hbm, blk_hbm, out_hbm, blkidx_v, rows_v, sem):
    wid = lax.axis_index("s") * SC_NC + lax.axis_index("c")
    base = wid * C_PER_W

    def chunk_body(ch, carry):
        off = base + ch * C_CHUNK
        pltpu.sync_copy(blk_hbm.at[pl.ds(off, C_CHUNK)], blkidx_v)
        pltpu.async_copy(tab_hbm.at[blkidx_v], rows_v, sem).wait()
        pltpu.sync_copy(rows_v, out_hbm.at[pl.ds(off, C_CHUNK)])
        return carry

    lax.fori_loop(0, C_PER_W // C_CHUNK, chunk_body, 0)


NCAND = K_TOP * BLKW

def _topk_body(c_ref, e_ref, x_ref, gw_ref, gb_ref,
               topi_ref, wts_ref, c_s):
    c_s[...] = c_ref[...]
    big = jnp.int32(NUM_SLOTS)
    iok = lax.broadcasted_iota(jnp.int32, (QB2, K_TOP), 1)

    def pass_body(j, carry):
        ti, tvv = carry
        c = c_s[...]
        e = e_ref[...]
        cur = jnp.max(c, axis=1, keepdims=True)
        gid = jnp.min(jnp.where(c == cur, e, big), axis=1, keepdims=True)
        ti = jnp.where(iok == j, gid, ti)
        tvv = jnp.where(iok == j, cur, tvv)
        c_s[...] = jnp.where(e == gid, NEG, c)
        return ti, tvv

    ti, tv = lax.fori_loop(
        0, K_TOP, pass_body,
        (jnp.zeros((QB2, K_TOP), jnp.int32),
         jnp.zeros((QB2, K_TOP), jnp.float32)))
    topi_ref[...] = ti
    a = jnp.exp((tv - tv[:, 0:1]) / jnp.float32(TAU))
    attn = a / jnp.sum(a, axis=1, keepdims=True)
    glin = lax.dot_general(x_ref[...], gw_ref[...], (((1,), (1,)), ((), ())),
                           preferred_element_type=jnp.float32)[:, 0:1]
    gate = jax.nn.sigmoid(glin + gb_ref[0, 0])
    w = attn * gate * jnp.float32(ALPHA)
    for j in range(K_TOP):
        wts_ref[:, j * 16:(j + 1) * 16] = jnp.broadcast_to(
            w[:, j:j + 1], (QB2, 16))


E_PER_W = SEQ // SC_NW
DCH = D_MODEL // 16

def _combine_body(vals_hbm, topi_hbm, wts_hbm, out_hbm,
                  idx_v, w_v, rows_v, orow_v, sem):
    wid = lax.axis_index("s") * SC_NC + lax.axis_index("c")
    base = wid * E_PER_W

    def row_body(i, carry):
        r = base + i
        pltpu.sync_copy(topi_hbm.at[r], idx_v)
        pltpu.sync_copy(wts_hbm.at[r], w_v)
        pltpu.async_copy(vals_hbm.at[idx_v], rows_v, sem).wait()
        wb = [w_v[pl.ds(k * 16, 16)] for k in range(K_TOP)]
        for dc in range(DCH):
            acc = wb[0] * rows_v[0, pl.ds(dc * 16, 16)]
            for k in range(1, K_TOP):
                acc = acc + wb[k] * rows_v[k, pl.ds(dc * 16, 16)]
            orow_v[pl.ds(dc * 16, 16)] = acc
        pltpu.sync_copy(orow_v, out_hbm.at[r])
        return carry

    lax.fori_loop(0, E_PER_W, row_body, 0)


@jax.jit
def kernel(x, keys, vals, W_q, gate_w, gate_b):
    B, L, D = x.shape
    x2d = x.reshape(L, D)

    qn = pl.pallas_call(
        _qn_body,
        out_shape=jax.ShapeDtypeStruct((L, D), jnp.float32),
        in_specs=[pl.BlockSpec((L, D), lambda: (0, 0)),
                  pl.BlockSpec((D, D), lambda: (0, 0))],
        out_specs=pl.BlockSpec((L, D), lambda: (0, 0)),
    )(x2d, W_q)

    nk = NUM_SLOTS // KB
    nq = L // QB
    scores, pooled = pl.pallas_call(
        _scores_body,
        grid=(nk, nq),
        out_shape=[jax.ShapeDtypeStruct((L, NUM_SLOTS), jnp.float32),
                   jax.ShapeDtypeStruct((nk, L, KB // GRP), jnp.float32)],
        in_specs=[pl.BlockSpec((KB, D), lambda k, q: (k, 0)),
                  pl.BlockSpec((QB, D), lambda k, q: (q, 0))],
        out_specs=[pl.BlockSpec((QB, KB), lambda k, q: (q, k)),
                   pl.BlockSpec((1, QB, KB // GRP), lambda k, q: (k, q, 0))],
        scratch_shapes=[pltpu.VMEM((KB, D), jnp.float32)],
    )(keys, qn)

    blkidx, eids = pl.pallas_call(
        _grpsel_body,
        grid=(L // QB2,),
        out_shape=[jax.ShapeDtypeStruct((L, K_TOP), jnp.int32),
                   jax.ShapeDtypeStruct((L, NCAND), jnp.int32)],
        in_specs=[pl.BlockSpec((NUM_SLOTS // KB, QB2, KB // GRP),
                               lambda i: (0, i, 0))],
        out_specs=[pl.BlockSpec((QB2, K_TOP), lambda i: (i, 0)),
                   pl.BlockSpec((QB2, NCAND), lambda i: (i, 0))],
        scratch_shapes=[pltpu.VMEM((QB2, NGRP), jnp.float32)],
    )(pooled)

    mesh = plsc.VectorSubcoreMesh(core_axis_name="c", subcore_axis_name="s")
    cand = pl.kernel(
        _cand_gather_body,
        out_type=jax.ShapeDtypeStruct((C_ROWS, BLKW), jnp.float32),
        mesh=mesh,
        scratch_types=[pltpu.VMEM((C_CHUNK,), jnp.int32),
                       pltpu.VMEM((C_CHUNK, BLKW), jnp.float32),
                       pltpu.SemaphoreType.DMA],
    )(scores.reshape(L * NBLK, BLKW), blkidx.reshape(C_ROWS))

    gw8 = jnp.broadcast_to(gate_w.reshape(1, D), (8, D))
    gb8 = jnp.broadcast_to(gate_b.reshape(1, 1), (8, 128))
    topi, wts = pl.pallas_call(
        _topk_body,
        grid=(L // QB2,),
        out_shape=[jax.ShapeDtypeStruct((L, K_TOP), jnp.int32),
                   jax.ShapeDtypeStruct((L, K_TOP * 16), jnp.float32)],
        in_specs=[pl.BlockSpec((QB2, NCAND), lambda i: (i, 0)),
                  pl.BlockSpec((QB2, NCAND), lambda i: (i, 0)),
                  pl.BlockSpec((QB2, D), lambda i: (i, 0)),
                  pl.BlockSpec((8, D), lambda i: (0, 0)),
                  pl.BlockSpec((8, 128), lambda i: (0, 0))],
        out_specs=[pl.BlockSpec((QB2, K_TOP), lambda i: (i, 0)),
                   pl.BlockSpec((QB2, K_TOP * 16), lambda i: (i, 0))],
        scratch_shapes=[pltpu.VMEM((QB2, NCAND), jnp.float32)],
    )(cand.reshape(L, NCAND), eids, x2d, gw8, gb8)

    out2d = pl.kernel(
        _combine_body,
        out_type=jax.ShapeDtypeStruct((L, D), jnp.float32),
        mesh=mesh,
        scratch_types=[pltpu.VMEM((K_TOP,), jnp.int32),
                       pltpu.VMEM((K_TOP * 16,), jnp.float32),
                       pltpu.VMEM((K_TOP, D), jnp.float32),
                       pltpu.VMEM((D,), jnp.float32),
                       pltpu.SemaphoreType.DMA],
    )(vals, topi, wts)

    return out2d.reshape(B, L, D)


# speedup vs baseline: 39.0950x; 1.0652x over previous
import functools

import jax
import jax.numpy as jnp
from jax import lax
from jax.experimental import pallas as pl
from jax.experimental.pallas import tpu as pltpu
from jax.experimental.pallas import tpu_sc as plsc

D_MODEL = 512
NUM_SLOTS = 65536
SEQ = 2048
K_TOP = 32
GRP = 128
NGRP = NUM_SLOTS // GRP
ALPHA = 1.0
TAU = 10.0
NEG = -3.0e38

SC_NC = 2
SC_NS = 16
SC_NW = SC_NC * SC_NS

def _qn_body(x_ref, w_ref, qn_ref):
    q = lax.dot_general(x_ref[...], w_ref[...], (((1,), (1,)), ((), ())),
                        preferred_element_type=jnp.float32)
    n = jnp.sqrt(jnp.sum(q * q, axis=1, keepdims=True))
    qn_ref[...] = q / jnp.maximum(n, 1e-12)


QB = 256
KB = 4096

def _scores_body(keys_ref, qn_ref, s_ref, pooled_ref, knb_ref):
    @pl.when(pl.program_id(1) == 0)
    def _():
        kb = keys_ref[...]
        n = jnp.sqrt(jnp.sum(kb * kb, axis=1, keepdims=True))
        knb_ref[...] = kb / jnp.maximum(n, 1e-12)

    s = lax.dot_general(qn_ref[...], knb_ref[...], (((1,), (1,)), ((), ())),
                        preferred_element_type=jnp.float32)
    s = s / jnp.float32(D_MODEL ** 0.5)
    s_ref[...] = s
    pooled_ref[0] = jnp.max(s.reshape(QB, KB // GRP, GRP), axis=-1)


QB2 = 256

def _grpsel_body(pooled_ref, blkidx_ref, eids_ref, p_ref):
    p_ref[...] = pooled_ref[...].transpose(1, 0, 2).reshape(QB2, NGRP)
    io = lax.broadcasted_iota(jnp.int32, (QB2, NGRP), 1)
    iob = lax.broadcasted_iota(jnp.int32, (QB2, BLKW), 1)
    iok = lax.broadcasted_iota(jnp.int32, (QB2, K_TOP), 1)
    rows = pl.program_id(0) * QB2 + lax.broadcasted_iota(jnp.int32, (QB2, 1), 0)

    def pass_body(j, bi):
        p = p_ref[...]
        cur = jnp.max(p, axis=1, keepdims=True)
        g = jnp.min(jnp.where(p == cur, io, NGRP), axis=1, keepdims=True)
        bi = jnp.where(iok == j, rows * NBLK + g, bi)
        blk = g
        eids_ref[:, pl.ds(j * BLKW, BLKW)] = blk * BLKW + iob
        p_ref[...] = jnp.where(io == g, NEG, p)
        return bi

    blkidx_ref[...] = lax.fori_loop(
        0, K_TOP, pass_body, jnp.zeros((QB2, K_TOP), jnp.int32))


BLKW = 128
NBLK = NUM_SLOTS // BLKW
C_ROWS = SEQ * K_TOP
C_PER_W = C_ROWS // SC_NW
C_CHUNK = 128

def _cand_gather_body(tab_hbm, blk_hbm, out_hbm, blkidx_v, rows_v, sem):
    wid = lax.axis_index("s") * SC_NC + lax.axis_index("c")
    base = wid * C_PER_W

    def chunk_body(ch, carry):
        off = base + ch * C_CHUNK
        pltpu.sync_copy(blk_hbm.at[pl.ds(off, C_CHUNK)], blkidx_v)
        pltpu.async_copy(tab_hbm.at[blkidx_v], rows_v, sem).wait()
        pltpu.sync_copy(rows_v, out_hbm.at[pl.ds(off, C_CHUNK)])
        return carry

    lax.fori_loop(0, C_PER_W // C_CHUNK, chunk_body, 0)


NCAND = K_TOP * BLKW

def _topk_body(c_ref, e_ref, x_ref, gw_ref, gb_ref,
               topi_ref, wts_ref, c_s):
    c_s[...] = c_ref[...]
    big = jnp.int32(NUM_SLOTS)
    iok = lax.broadcasted_iota(jnp.int32, (QB2, K_TOP), 1)

    def pass_body(j, carry):
        ti, tvv = carry
        c = c_s[...]
        e = e_ref[...]
        cur = jnp.max(c, axis=1, keepdims=True)
        gid = jnp.min(jnp.where(c == cur, e, big), axis=1, keepdims=True)
        ti = jnp.where(iok == j, gid, ti)
        tvv = jnp.where(iok == j, cur, tvv)
        c_s[...] = jnp.where(e == gid, NEG, c)
        return ti, tvv

    ti, tv = lax.fori_loop(
        0, K_TOP, pass_body,
        (jnp.zeros((QB2, K_TOP), jnp.int32),
         jnp.zeros((QB2, K_TOP), jnp.float32)))
    topi_ref[...] = ti
    a = jnp.exp((tv - tv[:, 0:1]) / jnp.float32(TAU))
    attn = a / jnp.sum(a, axis=1, keepdims=True)
    glin = lax.dot_general(x_ref[...], gw_ref[...], (((1,), (1,)), ((), ())),
                           preferred_element_type=jnp.float32)[:, 0:1]
    gate = jax.nn.sigmoid(glin + gb_ref[0, 0])
    w = attn * gate * jnp.float32(ALPHA)
    for j in range(K_TOP):
        wts_ref[:, j * 16:(j + 1) * 16] = jnp.broadcast_to(
            w[:, j:j + 1], (QB2, 16))


E_PER_W = SEQ // SC_NW
DCH = D_MODEL // 16

def _combine_body(vals_hbm, topi_hbm, wts_hbm, out_hbm,
                  idx_all, w_all, rows0, rows1, oall, sem0, sem1):
    wid = lax.axis_index("s") * SC_NC + lax.axis_index("c")
    base = wid * E_PER_W
    pltpu.sync_copy(topi_hbm.at[pl.ds(base, E_PER_W)], idx_all)
    pltpu.sync_copy(wts_hbm.at[pl.ds(base, E_PER_W)], w_all)
    pltpu.async_copy(vals_hbm.at[idx_all.at[0]], rows0, sem0)

    def compute_row(rows_v, r):
        wb = [w_all[r, pl.ds(k * 16, 16)] for k in range(K_TOP)]
        for dc in range(DCH):
            acc = wb[0] * rows_v[0, pl.ds(dc * 16, 16)]
            for k in range(1, K_TOP):
                acc = acc + wb[k] * rows_v[k, pl.ds(dc * 16, 16)]
            oall[r, pl.ds(dc * 16, 16)] = acc

    def body(i, carry):
        r0 = 2 * i
        pltpu.async_copy(vals_hbm.at[idx_all.at[r0 + 1]], rows1, sem1)
        pltpu.make_async_copy(vals_hbm.at[pl.ds(0, K_TOP)], rows0, sem0).wait()
        compute_row(rows0, r0)

        @pl.when(i < E_PER_W // 2 - 1)
        def _():
            pltpu.async_copy(vals_hbm.at[idx_all.at[r0 + 2]], rows0, sem0)

        pltpu.make_async_copy(vals_hbm.at[pl.ds(0, K_TOP)], rows1, sem1).wait()
        compute_row(rows1, r0 + 1)
        return carry

    lax.fori_loop(0, E_PER_W // 2, body, 0)
    pltpu.sync_copy(oall, out_hbm.at[pl.ds(base, E_PER_W)])


@jax.jit
def kernel(x, keys, vals, W_q, gate_w, gate_b):
    B, L, D = x.shape
    x2d = x.reshape(L, D)

    qn = pl.pallas_call(
        _qn_body,
        out_shape=jax.ShapeDtypeStruct((L, D), jnp.float32),
        in_specs=[pl.BlockSpec((L, D), lambda: (0, 0)),
                  pl.BlockSpec((D, D), lambda: (0, 0))],
        out_specs=pl.BlockSpec((L, D), lambda: (0, 0)),
    )(x2d, W_q)

    nk = NUM_SLOTS // KB
    nq = L // QB
    scores, pooled = pl.pallas_call(
        _scores_body,
        grid=(nk, nq),
        out_shape=[jax.ShapeDtypeStruct((L, NUM_SLOTS), jnp.float32),
                   jax.ShapeDtypeStruct((nk, L, KB // GRP), jnp.float32)],
        in_specs=[pl.BlockSpec((KB, D), lambda k, q: (k, 0)),
                  pl.BlockSpec((QB, D), lambda k, q: (q, 0))],
        out_specs=[pl.BlockSpec((QB, KB), lambda k, q: (q, k)),
                   pl.BlockSpec((1, QB, KB // GRP), lambda k, q: (k, q, 0))],
        scratch_shapes=[pltpu.VMEM((KB, D), jnp.float32)],
    )(keys, qn)

    blkidx, eids = pl.pallas_call(
        _grpsel_body,
        grid=(L // QB2,),
        out_shape=[jax.ShapeDtypeStruct((L, K_TOP), jnp.int32),
                   jax.ShapeDtypeStruct((L, NCAND), jnp.int32)],
        in_specs=[pl.BlockSpec((NUM_SLOTS // KB, QB2, KB // GRP),
                               lambda i: (0, i, 0))],
        out_specs=[pl.BlockSpec((QB2, K_TOP), lambda i: (i, 0)),
                   pl.BlockSpec((QB2, NCAND), lambda i: (i, 0))],
        scratch_shapes=[pltpu.VMEM((QB2, NGRP), jnp.float32)],
    )(pooled)

    mesh = plsc.VectorSubcoreMesh(core_axis_name="c", subcore_axis_name="s")
    cand = pl.kernel(
        _cand_gather_body,
        out_type=jax.ShapeDtypeStruct((C_ROWS, BLKW), jnp.float32),
        mesh=mesh,
        scratch_types=[pltpu.VMEM((C_CHUNK,), jnp.int32),
                       pltpu.VMEM((C_CHUNK, BLKW), jnp.float32),
                       pltpu.SemaphoreType.DMA],
    )(scores.reshape(L * NBLK, BLKW), blkidx.reshape(C_ROWS))

    gw8 = jnp.broadcast_to(gate_w.reshape(1, D), (8, D))
    gb8 = jnp.broadcast_to(gate_b.reshape(1, 1), (8, 128))
    topi, wts = pl.pallas_call(
        _topk_body,
        grid=(L // QB2,),
        out_shape=[jax.ShapeDtypeStruct((L, K_TOP), jnp.int32),
                   jax.ShapeDtypeStruct((L, K_TOP * 16), jnp.float32)],
        in_specs=[pl.BlockSpec((QB2, NCAND), lambda i: (i, 0)),
                  pl.BlockSpec((QB2, NCAND), lambda i: (i, 0)),
                  pl.BlockSpec((QB2, D), lambda i: (i, 0)),
                  pl.BlockSpec((8, D), lambda i: (0, 0)),
                  pl.BlockSpec((8, 128), lambda i: (0, 0))],
        out_specs=[pl.BlockSpec((QB2, K_TOP), lambda i: (i, 0)),
                   pl.BlockSpec((QB2, K_TOP * 16), lambda i: (i, 0))],
        scratch_shapes=[pltpu.VMEM((QB2, NCAND), jnp.float32)],
    )(cand.reshape(L, NCAND), eids, x2d, gw8, gb8)

    out2d = pl.kernel(
        _combine_body,
        out_type=jax.ShapeDtypeStruct((L, D), jnp.float32),
        mesh=mesh,
        scratch_types=[pltpu.VMEM((E_PER_W, K_TOP), jnp.int32),
                       pltpu.VMEM((E_PER_W, K_TOP * 16), jnp.float32),
                       pltpu.VMEM((K_TOP, D), jnp.float32),
                       pltpu.VMEM((K_TOP, D), jnp.float32),
                       pltpu.VMEM((E_PER_W, D), jnp.float32),
                       pltpu.SemaphoreType.DMA,
                       pltpu.SemaphoreType.DMA],
    )(vals, topi, wts)

    return out2d.reshape(B, L, D)


# speedup vs baseline: 40.9316x; 1.0470x over previous
import functools

import jax
import jax.numpy as jnp
from jax import lax
from jax.experimental import pallas as pl
from jax.experimental.pallas import tpu as pltpu
from jax.experimental.pallas import tpu_sc as plsc

D_MODEL = 512
NUM_SLOTS = 65536
SEQ = 2048
K_TOP = 32
GRP = 128
NGRP = NUM_SLOTS // GRP
ALPHA = 1.0
TAU = 10.0
NEG = -3.0e38

SC_NC = 2
SC_NS = 16
SC_NW = SC_NC * SC_NS

def _qn_body(x_ref, w_ref, qn_ref):
    q = lax.dot_general(x_ref[...], w_ref[...], (((1,), (1,)), ((), ())),
                        preferred_element_type=jnp.float32)
    n = jnp.sqrt(jnp.sum(q * q, axis=1, keepdims=True))
    qn_ref[...] = q / jnp.maximum(n, 1e-12)


QB = 512
KB = 4096

def _scores_body(keys_ref, qn_ref, s_ref, pooled_ref, knb_ref):
    @pl.when(pl.program_id(1) == 0)
    def _():
        kb = keys_ref[...]
        n = jnp.sqrt(jnp.sum(kb * kb, axis=1, keepdims=True))
        knb_ref[...] = kb / jnp.maximum(n, 1e-12)

    s = lax.dot_general(qn_ref[...], knb_ref[...], (((1,), (1,)), ((), ())),
                        preferred_element_type=jnp.float32)
    s = s / jnp.float32(D_MODEL ** 0.5)
    s_ref[...] = s
    pooled_ref[0] = jnp.max(s.reshape(QB, KB // GRP, GRP), axis=-1)


QB2 = 512

def _grpsel_body(pooled_ref, blkidx_ref, eids_ref, p_ref):
    p_ref[...] = pooled_ref[...].transpose(1, 0, 2).reshape(QB2, NGRP)
    io = lax.broadcasted_iota(jnp.int32, (QB2, NGRP), 1)
    iob = lax.broadcasted_iota(jnp.int32, (QB2, BLKW), 1)
    iok = lax.broadcasted_iota(jnp.int32, (QB2, K_TOP), 1)
    rows = pl.program_id(0) * QB2 + lax.broadcasted_iota(jnp.int32, (QB2, 1), 0)

    def pass_body(j, bi):
        p = p_ref[...]
        cur = jnp.max(p, axis=1, keepdims=True)
        g = jnp.min(jnp.where(p == cur, io, NGRP), axis=1, keepdims=True)
        bi = jnp.where(iok == j, rows * NBLK + g, bi)
        blk = g
        eids_ref[:, pl.ds(j * BLKW, BLKW)] = blk * BLKW + iob
        p_ref[...] = jnp.where(io == g, NEG, p)
        return bi

    blkidx_ref[...] = lax.fori_loop(
        0, K_TOP, pass_body, jnp.zeros((QB2, K_TOP), jnp.int32))


BLKW = 128
NBLK = NUM_SLOTS // BLKW
C_ROWS = SEQ * K_TOP
C_PER_W = C_ROWS // SC_NW
C_CHUNK = 128

def _cand_gather_body(tab_hbm, blk_hbm, out_hbm, blkidx_v, rows_v, sem):
    wid = lax.axis_index("s") * SC_NC + lax.axis_index("c")
    base = wid * C_PER_W

    def chunk_body(ch, carry):
        off = base + ch * C_CHUNK
        pltpu.sync_copy(blk_hbm.at[pl.ds(off, C_CHUNK)], blkidx_v)
        pltpu.async_copy(tab_hbm.at[blkidx_v], rows_v, sem).wait()
        pltpu.sync_copy(rows_v, out_hbm.at[pl.ds(off, C_CHUNK)])
        return carry

    lax.fori_loop(0, C_PER_W // C_CHUNK, chunk_body, 0)


NCAND = K_TOP * BLKW

def _topk_body(c_ref, e_ref, x_ref, gw_ref, gb_ref,
               topi_ref, wts_ref, c_s):
    c_s[...] = c_ref[...]
    big = jnp.int32(NUM_SLOTS)
    iok = lax.broadcasted_iota(jnp.int32, (QB2, K_TOP), 1)

    def pass_body(j, carry):
        ti, tvv = carry
        c = c_s[...]
        e = e_ref[...]
        cur = jnp.max(c, axis=1, keepdims=True)
        gid = jnp.min(jnp.where(c == cur, e, big), axis=1, keepdims=True)
        ti = jnp.where(iok == j, gid, ti)
        tvv = jnp.where(iok == j, cur, tvv)
        c_s[...] = jnp.where(e == gid, NEG, c)
        return ti, tvv

    ti, tv = lax.fori_loop(
        0, K_TOP, pass_body,
        (jnp.zeros((QB2, K_TOP), jnp.int32),
         jnp.zeros((QB2, K_TOP), jnp.float32)))
    topi_ref[...] = ti
    a = jnp.exp((tv - tv[:, 0:1]) / jnp.float32(TAU))
    attn = a / jnp.sum(a, axis=1, keepdims=True)
    glin = lax.dot_general(x_ref[...], gw_ref[...], (((1,), (1,)), ((), ())),
                           preferred_element_type=jnp.float32)[:, 0:1]
    gate = jax.nn.sigmoid(glin + gb_ref[0, 0])
    w = attn * gate * jnp.float32(ALPHA)
    for j in range(K_TOP):
        wts_ref[:, j * 16:(j + 1) * 16] = jnp.broadcast_to(
            w[:, j:j + 1], (QB2, 16))


E_PER_W = SEQ // SC_NW
DCH = D_MODEL // 16

def _combine_body(vals_hbm, topi_hbm, wts_hbm, out_hbm,
                  idx_all, w_all, rows0, rows1, oall, sem0, sem1):
    wid = lax.axis_index("s") * SC_NC + lax.axis_index("c")
    base = wid * E_PER_W
    pltpu.sync_copy(topi_hbm.at[pl.ds(base, E_PER_W)], idx_all)
    pltpu.sync_copy(wts_hbm.at[pl.ds(base, E_PER_W)], w_all)
    pltpu.async_copy(vals_hbm.at[idx_all.at[0]], rows0, sem0)

    def compute_row(rows_v, r):
        wb = [w_all[r, pl.ds(k * 16, 16)] for k in range(K_TOP)]
        for dc in range(DCH):
            acc = wb[0] * rows_v[0, pl.ds(dc * 16, 16)]
            for k in range(1, K_TOP):
                acc = acc + wb[k] * rows_v[k, pl.ds(dc * 16, 16)]
            oall[r, pl.ds(dc * 16, 16)] = acc

    def body(i, carry):
        r0 = 2 * i
        pltpu.async_copy(vals_hbm.at[idx_all.at[r0 + 1]], rows1, sem1)
        pltpu.make_async_copy(vals_hbm.at[pl.ds(0, K_TOP)], rows0, sem0).wait()
        compute_row(rows0, r0)

        @pl.when(i < E_PER_W // 2 - 1)
        def _():
            pltpu.async_copy(vals_hbm.at[idx_all.at[r0 + 2]], rows0, sem0)

        pltpu.make_async_copy(vals_hbm.at[pl.ds(0, K_TOP)], rows1, sem1).wait()
        compute_row(rows1, r0 + 1)
        return carry

    lax.fori_loop(0, E_PER_W // 2, body, 0)
    pltpu.sync_copy(oall, out_hbm.at[pl.ds(base, E_PER_W)])


@jax.jit
def kernel(x, keys, vals, W_q, gate_w, gate_b):
    B, L, D = x.shape
    x2d = x.reshape(L, D)

    qn = pl.pallas_call(
        _qn_body,
        out_shape=jax.ShapeDtypeStruct((L, D), jnp.float32),
        in_specs=[pl.BlockSpec((L, D), lambda: (0, 0)),
                  pl.BlockSpec((D, D), lambda: (0, 0))],
        out_specs=pl.BlockSpec((L, D), lambda: (0, 0)),
    )(x2d, W_q)

    nk = NUM_SLOTS // KB
    nq = L // QB
    scores, pooled = pl.pallas_call(
        _scores_body,
        grid=(nk, nq),
        out_shape=[jax.ShapeDtypeStruct((L, NUM_SLOTS), jnp.float32),
                   jax.ShapeDtypeStruct((nk, L, KB // GRP), jnp.float32)],
        in_specs=[pl.BlockSpec((KB, D), lambda k, q: (k, 0)),
                  pl.BlockSpec((QB, D), lambda k, q: (q, 0))],
        out_specs=[pl.BlockSpec((QB, KB), lambda k, q: (q, k)),
                   pl.BlockSpec((1, QB, KB // GRP), lambda k, q: (k, q, 0))],
        scratch_shapes=[pltpu.VMEM((KB, D), jnp.float32)],
    )(keys, qn)

    blkidx, eids = pl.pallas_call(
        _grpsel_body,
        grid=(L // QB2,),
        out_shape=[jax.ShapeDtypeStruct((L, K_TOP), jnp.int32),
                   jax.ShapeDtypeStruct((L, NCAND), jnp.int32)],
        in_specs=[pl.BlockSpec((NUM_SLOTS // KB, QB2, KB // GRP),
                               lambda i: (0, i, 0))],
        out_specs=[pl.BlockSpec((QB2, K_TOP), lambda i: (i, 0)),
                   pl.BlockSpec((QB2, NCAND), lambda i: (i, 0))],
        scratch_shapes=[pltpu.VMEM((QB2, NGRP), jnp.float32)],
    )(pooled)

    mesh = plsc.VectorSubcoreMesh(core_axis_name="c", subcore_axis_name="s")
    cand = pl.kernel(
        _cand_gather_body,
        out_type=jax.ShapeDtypeStruct((C_ROWS, BLKW), jnp.float32),
        mesh=mesh,
        scratch_types=[pltpu.VMEM((C_CHUNK,), jnp.int32),
                       pltpu.VMEM((C_CHUNK, BLKW), jnp.float32),
                       pltpu.SemaphoreType.DMA],
    )(scores.reshape(L * NBLK, BLKW), blkidx.reshape(C_ROWS))

    gw8 = jnp.broadcast_to(gate_w.reshape(1, D), (8, D))
    gb8 = jnp.broadcast_to(gate_b.reshape(1, 1), (8, 128))
    topi, wts = pl.pallas_call(
        _topk_body,
        grid=(L // QB2,),
        out_shape=[jax.ShapeDtypeStruct((L, K_TOP), jnp.int32),
                   jax.ShapeDtypeStruct((L, K_TOP * 16), jnp.float32)],
        in_specs=[pl.BlockSpec((QB2, NCAND), lambda i: (i, 0)),
                  pl.BlockSpec((QB2, NCAND), lambda i: (i, 0)),
                  pl.BlockSpec((QB2, D), lambda i: (i, 0)),
                  pl.BlockSpec((8, D), lambda i: (0, 0)),
                  pl.BlockSpec((8, 128), lambda i: (0, 0))],
        out_specs=[pl.BlockSpec((QB2, K_TOP), lambda i: (i, 0)),
                   pl.BlockSpec((QB2, K_TOP * 16), lambda i: (i, 0))],
        scratch_shapes=[pltpu.VMEM((QB2, NCAND), jnp.float32)],
    )(cand.reshape(L, NCAND), eids, x2d, gw8, gb8)

    out2d = pl.kernel(
        _combine_body,
        out_type=jax.ShapeDtypeStruct((L, D), jnp.float32),
        mesh=mesh,
        scratch_types=[pltpu.VMEM((E_PER_W, K_TOP), jnp.int32),
                       pltpu.VMEM((E_PER_W, K_TOP * 16), jnp.float32),
                       pltpu.VMEM((K_TOP, D), jnp.float32),
                       pltpu.VMEM((K_TOP, D), jnp.float32),
                       pltpu.VMEM((E_PER_W, D), jnp.float32),
                       pltpu.SemaphoreType.DMA,
                       pltpu.SemaphoreType.DMA],
    )(vals, topi, wts)

    return out2d.reshape(B, L, D)


# speedup vs baseline: 45.0720x; 1.1012x over previous
import functools

import jax
import jax.numpy as jnp
from jax import lax
from jax.experimental import pallas as pl
from jax.experimental.pallas import tpu as pltpu
from jax.experimental.pallas import tpu_sc as plsc

D_MODEL = 512
NUM_SLOTS = 65536
SEQ = 2048
K_TOP = 32
GRP = 128
NGRP = NUM_SLOTS // GRP
ALPHA = 1.0
TAU = 10.0
NEG = -3.0e38

SC_NC = 2
SC_NS = 16
SC_NW = SC_NC * SC_NS

def _qn_body(x_ref, w_ref, qn_ref):
    q = lax.dot_general(x_ref[...], w_ref[...], (((1,), (1,)), ((), ())),
                        preferred_element_type=jnp.float32)
    n = jnp.sqrt(jnp.sum(q * q, axis=1, keepdims=True))
    qn_ref[...] = q / jnp.maximum(n, 1e-12)


QB = 512
KB = 4096

def _scores_body(keys_ref, qn_ref, s_ref, pooled_ref, knb_ref):
    @pl.when(pl.program_id(1) == 0)
    def _():
        kb = keys_ref[...]
        n = jnp.sqrt(jnp.sum(kb * kb, axis=1, keepdims=True))
        knb_ref[...] = kb / jnp.maximum(n, 1e-12)

    s = lax.dot_general(qn_ref[...], knb_ref[...], (((1,), (1,)), ((), ())),
                        preferred_element_type=jnp.float32)
    s = s / jnp.float32(D_MODEL ** 0.5)
    s_ref[...] = s
    pooled_ref[0] = jnp.max(s.reshape(QB, KB // GRP, GRP), axis=-1)


QB2 = 512

def _grpsel_body(pooled_ref, blkidx_ref, eids_ref, p_ref):
    p_ref[...] = pooled_ref[...].transpose(1, 0, 2).reshape(QB2, NGRP)
    io = lax.broadcasted_iota(jnp.int32, (QB2, NGRP), 1)
    iob = lax.broadcasted_iota(jnp.int32, (QB2, BLKW), 1)
    iok = lax.broadcasted_iota(jnp.int32, (QB2, K_TOP), 1)
    rows = pl.program_id(0) * QB2 + lax.broadcasted_iota(jnp.int32, (QB2, 1), 0)

    def pass_body(j, bi):
        p = p_ref[...]
        cur = jnp.max(p, axis=1, keepdims=True)
        g = jnp.min(jnp.where(p == cur, io, NGRP), axis=1, keepdims=True)
        bi = jnp.where(iok == j, rows * NBLK + g, bi)
        blk = g
        eids_ref[:, pl.ds(j * BLKW, BLKW)] = blk * BLKW + iob
        p_ref[...] = jnp.where(io == g, NEG, p)
        return bi

    blkidx_ref[...] = lax.fori_loop(
        0, K_TOP, pass_body, jnp.zeros((QB2, K_TOP), jnp.int32))


BLKW = 128
NBLK = NUM_SLOTS // BLKW
C_ROWS = SEQ * K_TOP
C_PER_W = C_ROWS // SC_NW
C_CHUNK = 128

def _cand_gather_body(tab_hbm, blk_hbm, out_hbm, blkidx_v, rows_v, sem):
    wid = lax.axis_index("s") * SC_NC + lax.axis_index("c")
    base = wid * C_PER_W

    def chunk_body(ch, carry):
        off = base + ch * C_CHUNK
        pltpu.sync_copy(blk_hbm.at[pl.ds(off, C_CHUNK)], blkidx_v)
        pltpu.async_copy(tab_hbm.at[blkidx_v], rows_v, sem).wait()
        pltpu.sync_copy(rows_v, out_hbm.at[pl.ds(off, C_CHUNK)])
        return carry

    lax.fori_loop(0, C_PER_W // C_CHUNK, chunk_body, 0)


NCAND = K_TOP * BLKW

def _topk_body(c_ref, e_ref, x_ref, gw_ref, gb_ref,
               topi_ref, wts_ref, c_s):
    c_s[...] = c_ref[...]
    big = jnp.int32(NUM_SLOTS)
    iok = lax.broadcasted_iota(jnp.int32, (QB2, K_TOP), 1)

    def pass_body(j, carry):
        ti, tvv = carry
        c = c_s[...]
        e = e_ref[...]
        cur = jnp.max(c, axis=1, keepdims=True)
        gid = jnp.min(jnp.where(c == cur, e, big), axis=1, keepdims=True)
        ti = jnp.where(iok == j, gid, ti)
        tvv = jnp.where(iok == j, cur, tvv)
        c_s[...] = jnp.where(e == gid, NEG, c)
        return ti, tvv

    ti, tv = lax.fori_loop(
        0, K_TOP, pass_body,
        (jnp.zeros((QB2, K_TOP), jnp.int32),
         jnp.zeros((QB2, K_TOP), jnp.float32)))
    topi_ref[...] = ti
    a = jnp.exp((tv - tv[:, 0:1]) / jnp.float32(TAU))
    attn = a / jnp.sum(a, axis=1, keepdims=True)
    glin = lax.dot_general(x_ref[...], gw_ref[...], (((1,), (1,)), ((), ())),
                           preferred_element_type=jnp.float32)[:, 0:1]
    gate = jax.nn.sigmoid(glin + gb_ref[0, 0])
    w = attn * gate * jnp.float32(ALPHA)
    for j in range(K_TOP):
        wts_ref[:, j * 16:(j + 1) * 16] = jnp.broadcast_to(
            w[:, j:j + 1], (QB2, 16))


E_PER_W = SEQ // SC_NW
DCH = D_MODEL // 16

E_DEPTH = 3
E_LOOP = (E_PER_W - 1) // E_DEPTH

def _combine_body(vals_hbm, topi_hbm, wts_hbm, out_hbm,
                  idx_all, w_all, rows0, rows1, rows2,
                  sem0, sem1, sem2):
    wid = lax.axis_index("s") * SC_NC + lax.axis_index("c")
    base = wid * E_PER_W
    rows = [rows0, rows1, rows2]
    sems = [sem0, sem1, sem2]
    pltpu.sync_copy(topi_hbm.at[pl.ds(base, E_PER_W)], idx_all)
    pltpu.sync_copy(wts_hbm.at[pl.ds(base, E_PER_W)], w_all)
    for j in range(E_DEPTH - 1):
        pltpu.async_copy(vals_hbm.at[idx_all.at[j]], rows[j], sems[j])

    def compute_row(rows_v, r):
        wb = [w_all[r, pl.ds(k * 16, 16)] for k in range(K_TOP)]

        def dc_body(dc, carry):
            o = dc * 16
            acc = wb[0] * rows_v[0, pl.ds(o, 16)]
            for k in range(1, K_TOP):
                acc = acc + wb[k] * rows_v[k, pl.ds(o, 16)]
            w_all[r, pl.ds(o, 16)] = acc
            return carry

        lax.fori_loop(0, DCH, dc_body, 0)

    def body(i, carry):
        r0 = E_DEPTH * i
        pltpu.async_copy(vals_hbm.at[idx_all.at[r0 + 2]], rows[2], sems[2])
        for j in range(E_DEPTH):
            pltpu.make_async_copy(vals_hbm.at[pl.ds(0, K_TOP)],
                                  rows[j], sems[j]).wait()
            compute_row(rows[j], r0 + j)

            @pl.when(r0 + E_DEPTH + j < E_PER_W)
            def _():
                pltpu.async_copy(vals_hbm.at[idx_all.at[r0 + E_DEPTH + j]],
                                 rows[j], sems[j])

        return carry

    lax.fori_loop(0, E_LOOP, body, 0)
    pltpu.make_async_copy(vals_hbm.at[pl.ds(0, K_TOP)], rows[0], sems[0]).wait()
    compute_row(rows[0], E_PER_W - 1)
    pltpu.sync_copy(w_all, out_hbm.at[pl.ds(base, E_PER_W)])


@jax.jit
def kernel(x, keys, vals, W_q, gate_w, gate_b):
    B, L, D = x.shape
    x2d = x.reshape(L, D)

    qn = pl.pallas_call(
        _qn_body,
        out_shape=jax.ShapeDtypeStruct((L, D), jnp.float32),
        in_specs=[pl.BlockSpec((L, D), lambda: (0, 0)),
                  pl.BlockSpec((D, D), lambda: (0, 0))],
        out_specs=pl.BlockSpec((L, D), lambda: (0, 0)),
    )(x2d, W_q)

    nk = NUM_SLOTS // KB
    nq = L // QB
    scores, pooled = pl.pallas_call(
        _scores_body,
        grid=(nk, nq),
        out_shape=[jax.ShapeDtypeStruct((L, NUM_SLOTS), jnp.float32),
                   jax.ShapeDtypeStruct((nk, L, KB // GRP), jnp.float32)],
        in_specs=[pl.BlockSpec((KB, D), lambda k, q: (k, 0)),
                  pl.BlockSpec((QB, D), lambda k, q: (q, 0))],
        out_specs=[pl.BlockSpec((QB, KB), lambda k, q: (q, k)),
                   pl.BlockSpec((1, QB, KB // GRP), lambda k, q: (k, q, 0))],
        scratch_shapes=[pltpu.VMEM((KB, D), jnp.float32)],
    )(keys, qn)

    blkidx, eids = pl.pallas_call(
        _grpsel_body,
        grid=(L // QB2,),
        out_shape=[jax.ShapeDtypeStruct((L, K_TOP), jnp.int32),
                   jax.ShapeDtypeStruct((L, NCAND), jnp.int32)],
        in_specs=[pl.BlockSpec((NUM_SLOTS // KB, QB2, KB // GRP),
                               lambda i: (0, i, 0))],
        out_specs=[pl.BlockSpec((QB2, K_TOP), lambda i: (i, 0)),
                   pl.BlockSpec((QB2, NCAND), lambda i: (i, 0))],
        scratch_shapes=[pltpu.VMEM((QB2, NGRP), jnp.float32)],
    )(pooled)

    mesh = plsc.VectorSubcoreMesh(core_axis_name="c", subcore_axis_name="s")
    cand = pl.kernel(
        _cand_gather_body,
        out_type=jax.ShapeDtypeStruct((C_ROWS, BLKW), jnp.float32),
        mesh=mesh,
        scratch_types=[pltpu.VMEM((C_CHUNK,), jnp.int32),
                       pltpu.VMEM((C_CHUNK, BLKW), jnp.float32),
                       pltpu.SemaphoreType.DMA],
    )(scores.reshape(L * NBLK, BLKW), blkidx.reshape(C_ROWS))

    gw8 = jnp.broadcast_to(gate_w.reshape(1, D), (8, D))
    gb8 = jnp.broadcast_to(gate_b.reshape(1, 1), (8, 128))
    topi, wts = pl.pallas_call(
        _topk_body,
        grid=(L // QB2,),
        out_shape=[jax.ShapeDtypeStruct((L, K_TOP), jnp.int32),
                   jax.ShapeDtypeStruct((L, K_TOP * 16), jnp.float32)],
        in_specs=[pl.BlockSpec((QB2, NCAND), lambda i: (i, 0)),
                  pl.BlockSpec((QB2, NCAND), lambda i: (i, 0)),
                  pl.BlockSpec((QB2, D), lambda i: (i, 0)),
                  pl.BlockSpec((8, D), lambda i: (0, 0)),
                  pl.BlockSpec((8, 128), lambda i: (0, 0))],
        out_specs=[pl.BlockSpec((QB2, K_TOP), lambda i: (i, 0)),
                   pl.BlockSpec((QB2, K_TOP * 16), lambda i: (i, 0))],
        scratch_shapes=[pltpu.VMEM((QB2, NCAND), jnp.float32)],
    )(cand.reshape(L, NCAND), eids, x2d, gw8, gb8)

    out2d = pl.kernel(
        _combine_body,
        out_type=jax.ShapeDtypeStruct((L, D), jnp.float32),
        mesh=mesh,
        scratch_types=[pltpu.VMEM((E_PER_W, K_TOP), jnp.int32),
                       pltpu.VMEM((E_PER_W, K_TOP * 16), jnp.float32),
                       pltpu.VMEM((K_TOP, D), jnp.float32),
                       pltpu.VMEM((K_TOP, D), jnp.float32),
                       pltpu.VMEM((K_TOP, D), jnp.float32),
                       pltpu.SemaphoreType.DMA,
                       pltpu.SemaphoreType.DMA,
                       pltpu.SemaphoreType.DMA],
    )(vals, topi, wts)

    return out2d.reshape(B, L, D)
